# Initial kernel scaffold; baseline (speedup 1.0000x reference)
import jax
import jax.numpy as jnp
from jax.experimental import pallas as pl


def kernel(x, global_bias, linear_weights, interaction_factors):
    raise NotImplementedError("write your pallas kernel here")


# speedup vs baseline: 2.1929x; 2.1929x over previous
import functools

import jax
import jax.numpy as jnp
from jax import lax
from jax.experimental import pallas as pl
from jax.experimental.pallas import tpu as pltpu
from jax.experimental.pallas import tpu_sc as plsc

_B = 16384
_F = 26
_D = 32
_NW = 32
_RPW = _B // _NW
_CHUNK = 64
_NCHUNK = _RPW // _CHUNK
_IDX_PER_CHUNK = _CHUNK * _F
_G = 128
_GPC = _IDX_PER_CHUNK // _G
_GPW = _RPW * _F // _G


def _fm_body(x_hbm, lw_hbm, emb_hbm, out_hbm,
             idx_v, rows_v, lin_v, out_v, sem):
    wid = lax.axis_index("s") * 2 + lax.axis_index("c")

    pltpu.sync_copy(x_hbm.at[wid], idx_v)

    lane = lax.broadcasted_iota(jnp.int32, (16,), 0)
    lmask = jnp.where(lane < _F - 16, 1.0, 0.0)

    def chunk_body(c, _):
        copies = []
        for j in range(_GPC):
            idx_row = idx_v.at[c * _GPC + j]
            copies.append(pltpu.async_copy(
                emb_hbm.at[idx_row], rows_v.at[pl.ds(j * _G, _G)], sem))
            copies.append(pltpu.async_copy(
                lw_hbm.at[idx_row], lin_v.at[pl.ds(j * _G, _G)], sem))
        for cp in copies:
            cp.wait()

        def grp_body(g, _):
            def row_body(i, acc):
                base = (g * 16 + i) * _F
                v0 = rows_v[base, pl.ds(0, 16)]
                v1 = rows_v[base, pl.ds(16, 16)]
                s0, q0 = v0, v0 * v0
                s1, q1 = v1, v1 * v1
                for f in range(1, _F):
                    v0 = rows_v[base + f, pl.ds(0, 16)]
                    v1 = rows_v[base + f, pl.ds(16, 16)]
                    s0 = s0 + v0
                    q0 = q0 + v0 * v0
                    s1 = s1 + v1
                    q1 = q1 + v1 * v1
                inter = (s0 * s0 - q0) + (s1 * s1 - q1)
                l0 = lin_v[pl.ds(base, 16)]
                l1 = lin_v[pl.ds(base + 16, 16)]
                t = inter * 0.5 + l0 + l1 * lmask
                return jnp.where(lane == i, jnp.sum(t), acc)

            acc = lax.fori_loop(0, 16, row_body, jnp.zeros((16,), jnp.float32))
            out_v[pl.ds(c * _CHUNK + g * 16, 16)] = acc
            return 0

        lax.fori_loop(0, _CHUNK // 16, grp_body, 0)
        return 0

    lax.fori_loop(0, _NCHUNK, chunk_body, 0)

    pltpu.sync_copy(out_v, out_hbm.at[pl.ds(wid * _RPW, _RPW)])


@jax.jit
def _fm_sc(x_grp, lw_flat, emb):
    mesh = plsc.VectorSubcoreMesh(core_axis_name="c", subcore_axis_name="s")
    return pl.kernel(
        _fm_body,
        out_type=jax.ShapeDtypeStruct((_B,), jnp.float32),
        mesh=mesh,
        compiler_params=pltpu.CompilerParams(
            needs_layout_passes=False, use_tc_tiling_on_sc=False),
        scratch_types=[
            pltpu.VMEM((_GPW, _G), jnp.int32),
            pltpu.VMEM((_IDX_PER_CHUNK, _D), jnp.float32),
            pltpu.VMEM((_IDX_PER_CHUNK + 16,), jnp.float32),
            pltpu.VMEM((_RPW,), jnp.float32),
            pltpu.SemaphoreType.DMA,
        ],
    )(x_grp, lw_flat, emb)


def kernel(x, global_bias, linear_weights, interaction_factors):
    x_grp = x.astype(jnp.int32).reshape(_NW, _GPW, _G)
    lw_flat = linear_weights.reshape(-1)
    out = _fm_sc(x_grp, lw_flat, interaction_factors)
    return out + global_bias[0]
